# baseline (speedup 1.0000x reference)
import functools

import jax
import jax.numpy as jnp
from jax import lax
from jax.experimental import pallas as pl
from jax.experimental.pallas import tpu as pltpu
from jax.experimental.pallas import tpu_sc as plsc

_NUM_REQS = 128
_LANES = 16
_REQS_PER_WORKER = 8
_NUM_WORKERS = _NUM_REQS // _REQS_PER_WORKER


def _last_loc_sc(table, pool_idx, prefix_lens):
    num_rows, stride = table.shape
    num_tokens = num_rows * stride
    assert stride & (stride - 1) == 0, "row stride must be a power of two"
    sh = stride.bit_length() - 1
    info = plsc.get_sparse_core_info()
    num_cores = info.num_cores
    mesh = plsc.VectorSubcoreMesh(core_axis_name="c", subcore_axis_name="s")

    @functools.partial(
        pl.kernel,
        mesh=mesh,
        compiler_params=pltpu.CompilerParams(
            use_tc_tiling_on_sc=False, needs_layout_passes=False),
        out_type=jax.ShapeDtypeStruct((_NUM_REQS,), jnp.int32),
        scratch_types=[
            pltpu.VMEM((_LANES,), jnp.int32),
            pltpu.VMEM((_LANES,), jnp.int32),
            pltpu.VMEM((_LANES,), jnp.int32),
            pltpu.VMEM((_LANES,), jnp.int32),
            pltpu.VMEM((_REQS_PER_WORKER, stride), jnp.int32),
            pltpu.SemaphoreType.DMA,
        ],
    )
    def body(table_hbm, pool_hbm, len_hbm, out_hbm,
             pool_v, len_v, idx_v, res_v, rows_v, sem):
        wid = lax.axis_index("s") * num_cores + lax.axis_index("c")

        @pl.when(wid < _NUM_WORKERS)
        def _():
            base = wid * _REQS_PER_WORKER
            pltpu.sync_copy(pool_hbm.at[pl.ds(base, _REQS_PER_WORKER)],
                            pool_v.at[pl.ds(0, _REQS_PER_WORKER)])
            pltpu.sync_copy(len_hbm.at[pl.ds(base, _REQS_PER_WORKER)],
                            len_v.at[pl.ds(0, _REQS_PER_WORKER)])
            lens = len_v[...]
            tok = pool_v[...] * stride + (lens - 1)
            tok = jnp.clip(tok, 0, num_tokens - 1)
            idx_v[...] = lax.shift_right_logical(tok, sh)
            col = lax.bitwise_and(tok, stride - 1)
            pltpu.async_copy(table_hbm.at[idx_v.at[pl.ds(0, _REQS_PER_WORKER)]],
                             rows_v, sem).wait()
            lane = lax.iota(jnp.int32, _LANES)
            acc = jnp.full((_LANES,), -1, jnp.int32)
            for j in range(_REQS_PER_WORKER):
                vals_j = plsc.load_gather(rows_v.at[j], [col])
                acc = jnp.where(lane == j, vals_j, acc)
            res_v[...] = jnp.where(lens > 0, acc, jnp.int32(-1))
            pltpu.sync_copy(res_v.at[pl.ds(0, _REQS_PER_WORKER)],
                            out_hbm.at[pl.ds(base, _REQS_PER_WORKER)])

    return body(table, pool_idx, prefix_lens)


def kernel(req_to_token, req_pool_indices_tensor, prefix_lens_tensor):
    table = req_to_token.astype(jnp.int32)
    pool = req_pool_indices_tensor.astype(jnp.int32)
    lens = prefix_lens_tensor.astype(jnp.int32)
    res = _last_loc_sc(table, pool, lens)
    return res.astype(req_to_token.dtype)


# speedup vs baseline: 4.9099x; 4.9099x over previous
import functools

import jax
import jax.numpy as jnp
from jax import lax
from jax.experimental import pallas as pl
from jax.experimental.pallas import tpu as pltpu
from jax.experimental.pallas import tpu_sc as plsc

_NUM_REQS = 128
_LANES = 16
_REQS_PER_WORKER = 8
_NUM_WORKERS = _NUM_REQS // _REQS_PER_WORKER


def _last_loc_sc(table, pool_idx, prefix_lens):
    num_rows, stride = table.shape
    num_tokens = num_rows * stride
    assert stride & (stride - 1) == 0, "row stride must be a power of two"
    sh = stride.bit_length() - 1
    info = plsc.get_sparse_core_info()
    num_cores = info.num_cores
    mesh = plsc.VectorSubcoreMesh(core_axis_name="c", subcore_axis_name="s")

    @functools.partial(
        pl.kernel,
        mesh=mesh,
        compiler_params=pltpu.CompilerParams(
            use_tc_tiling_on_sc=True, needs_layout_passes=False),
        out_type=jax.ShapeDtypeStruct((_NUM_REQS,), jnp.int32),
        scratch_types=[
            pltpu.VMEM((_LANES,), jnp.int32),
            pltpu.VMEM((_LANES,), jnp.int32),
            pltpu.VMEM((_LANES,), jnp.int32),
            pltpu.VMEM((_LANES,), jnp.int32),
            pltpu.VMEM((_REQS_PER_WORKER, stride), jnp.int32),
            pltpu.SemaphoreType.DMA,
        ],
    )
    def body(table_hbm, pool_hbm, len_hbm, out_hbm,
             pool_v, len_v, idx_v, res_v, rows_v, sem):
        wid = lax.axis_index("s") * num_cores + lax.axis_index("c")

        @pl.when(wid < _NUM_WORKERS)
        def _():
            base = wid * _REQS_PER_WORKER
            pltpu.sync_copy(pool_hbm.at[pl.ds(base, _REQS_PER_WORKER)],
                            pool_v.at[pl.ds(0, _REQS_PER_WORKER)])
            pltpu.sync_copy(len_hbm.at[pl.ds(base, _REQS_PER_WORKER)],
                            len_v.at[pl.ds(0, _REQS_PER_WORKER)])
            lens = len_v[...]
            tok = pool_v[...] * stride + (lens - 1)
            tok = jnp.clip(tok, 0, num_tokens - 1)
            idx_v[...] = lax.shift_right_logical(tok, sh)
            col = lax.bitwise_and(tok, stride - 1)
            pltpu.async_copy(table_hbm.at[idx_v.at[pl.ds(0, _REQS_PER_WORKER)]],
                             rows_v, sem).wait()
            lane = lax.iota(jnp.int32, _LANES)
            vals = plsc.load_gather(
                rows_v, [lax.bitwise_and(lane, _REQS_PER_WORKER - 1), col])
            res_v[...] = jnp.where(lens > 0, vals, jnp.int32(-1))
            pltpu.sync_copy(res_v.at[pl.ds(0, _REQS_PER_WORKER)],
                            out_hbm.at[pl.ds(base, _REQS_PER_WORKER)])

    return body(table, pool_idx, prefix_lens)


def kernel(req_to_token, req_pool_indices_tensor, prefix_lens_tensor):
    table = req_to_token.astype(jnp.int32)
    pool = req_pool_indices_tensor.astype(jnp.int32)
    lens = prefix_lens_tensor.astype(jnp.int32)
    res = _last_loc_sc(table, pool, lens)
    return res.astype(req_to_token.dtype)


# speedup vs baseline: 5.6311x; 1.1469x over previous
import functools

import jax
import jax.numpy as jnp
from jax import lax
from jax.experimental import pallas as pl
from jax.experimental.pallas import tpu as pltpu
from jax.experimental.pallas import tpu_sc as plsc

_NUM_REQS = 128
_LANES = 16
_REQS_PER_WORKER = 8
_NUM_WORKERS = _NUM_REQS // _REQS_PER_WORKER


def _last_loc_sc(table, pool_idx, prefix_lens):
    num_rows, stride = table.shape
    num_tokens = num_rows * stride
    assert stride & (stride - 1) == 0, "row stride must be a power of two"
    sh = stride.bit_length() - 1
    info = plsc.get_sparse_core_info()
    num_cores = info.num_cores
    mesh = plsc.VectorSubcoreMesh(core_axis_name="c", subcore_axis_name="s")

    @functools.partial(
        pl.kernel,
        mesh=mesh,
        compiler_params=pltpu.CompilerParams(
            use_tc_tiling_on_sc=True, needs_layout_passes=False),
        out_type=jax.ShapeDtypeStruct((_NUM_REQS,), jnp.int32),
        scratch_types=[
            pltpu.VMEM((_LANES,), jnp.int32),
            pltpu.VMEM((_LANES,), jnp.int32),
            pltpu.VMEM((_LANES,), jnp.int32),
            pltpu.VMEM((_REQS_PER_WORKER * 8, 128), jnp.int32),
            pltpu.SemaphoreType.DMA,
            pltpu.SemaphoreType.DMA,
        ],
    )
    def body(table_hbm, pool_hbm, len_hbm, out_hbm,
             pool_v, len_v, res_v, tiles_v, sem, sem2):
        wid = lax.axis_index("s") * num_cores + lax.axis_index("c")

        @pl.when(wid < _NUM_WORKERS)
        def _():
            base = wid * _REQS_PER_WORKER
            cp1 = pltpu.async_copy(
                pool_hbm.at[pl.ds(base, _REQS_PER_WORKER)],
                pool_v.at[pl.ds(0, _REQS_PER_WORKER)], sem)
            cp2 = pltpu.async_copy(
                len_hbm.at[pl.ds(base, _REQS_PER_WORKER)],
                len_v.at[pl.ds(0, _REQS_PER_WORKER)], sem)
            cp1.wait()
            cp2.wait()
            lens = len_v[...]
            tok = pool_v[...] * stride + (lens - 1)
            tok = jnp.clip(tok, 0, num_tokens - 1)
            row8_v = lax.bitwise_and(
                lax.shift_right_logical(tok, sh), jnp.int32(~7))
            colb_v = lax.bitwise_and(tok, jnp.int32((stride - 1) & ~127))
            lane = lax.iota(jnp.int32, _LANES)
            copies = []
            for j in range(_REQS_PER_WORKER):
                sel = lane == j
                r8 = pl.multiple_of(
                    jnp.max(jnp.where(sel, row8_v, jnp.int32(0))), 8)
                cb = pl.multiple_of(
                    jnp.max(jnp.where(sel, colb_v, jnp.int32(0))), 128)
                copies.append(pltpu.async_copy(
                    table_hbm.at[pl.ds(r8, 8), pl.ds(cb, 128)],
                    tiles_v.at[pl.ds(j * 8, 8), pl.ds(0, 128)], sem2))
            for cp in copies:
                cp.wait()
            row_in = lax.bitwise_and(
                lax.shift_right_logical(tok, sh), jnp.int32(7))
            col_in = lax.bitwise_and(tok, jnp.int32(127))
            tile_row = lax.bitwise_and(lane, _REQS_PER_WORKER - 1) * 8 + row_in
            vals = plsc.load_gather(tiles_v, [tile_row, col_in])
            res_v[...] = jnp.where(lens > 0, vals, jnp.int32(-1))
            pltpu.sync_copy(res_v.at[pl.ds(0, _REQS_PER_WORKER)],
                            out_hbm.at[pl.ds(base, _REQS_PER_WORKER)])

    return body(table, pool_idx, prefix_lens)


def kernel(req_to_token, req_pool_indices_tensor, prefix_lens_tensor):
    table = req_to_token.astype(jnp.int32)
    pool = req_pool_indices_tensor.astype(jnp.int32)
    lens = prefix_lens_tensor.astype(jnp.int32)
    res = _last_loc_sc(table, pool, lens)
    return res.astype(req_to_token.dtype)


# speedup vs baseline: 6.1788x; 1.0973x over previous
import functools

import jax
import jax.numpy as jnp
from jax import lax
from jax.experimental import pallas as pl
from jax.experimental.pallas import tpu as pltpu
from jax.experimental.pallas import tpu_sc as plsc

_NUM_REQS = 128
_LANES = 16
_REQS_PER_WORKER = 8
_NUM_WORKERS = _NUM_REQS // _REQS_PER_WORKER


def _last_loc_sc(table, pool_idx, prefix_lens):
    num_rows, stride = table.shape
    num_tokens = num_rows * stride
    assert stride & (stride - 1) == 0, "row stride must be a power of two"
    sh = stride.bit_length() - 1
    num_cores = 1
    mesh = plsc.VectorSubcoreMesh(
        core_axis_name="c", subcore_axis_name="s", num_cores=num_cores)

    @functools.partial(
        pl.kernel,
        mesh=mesh,
        compiler_params=pltpu.CompilerParams(
            use_tc_tiling_on_sc=True, needs_layout_passes=False),
        out_type=jax.ShapeDtypeStruct((_NUM_REQS,), jnp.int32),
        scratch_types=[
            pltpu.VMEM((_LANES,), jnp.int32),
            pltpu.VMEM((_LANES,), jnp.int32),
            pltpu.VMEM((_LANES,), jnp.int32),
            pltpu.VMEM((_REQS_PER_WORKER * 8, 128), jnp.int32),
            pltpu.SemaphoreType.DMA,
            pltpu.SemaphoreType.DMA,
        ],
    )
    def body(table_hbm, pool_hbm, len_hbm, out_hbm,
             pool_v, len_v, res_v, tiles_v, sem, sem2):
        wid = lax.axis_index("s") * num_cores + lax.axis_index("c")

        @pl.when(wid < _NUM_WORKERS)
        def _():
            base = wid * _REQS_PER_WORKER
            cp1 = pltpu.async_copy(
                pool_hbm.at[pl.ds(base, _REQS_PER_WORKER)],
                pool_v.at[pl.ds(0, _REQS_PER_WORKER)], sem)
            cp2 = pltpu.async_copy(
                len_hbm.at[pl.ds(base, _REQS_PER_WORKER)],
                len_v.at[pl.ds(0, _REQS_PER_WORKER)], sem)
            cp1.wait()
            cp2.wait()
            lens = len_v[...]
            tok = pool_v[...] * stride + (lens - 1)
            tok = jnp.clip(tok, 0, num_tokens - 1)
            row8_v = lax.bitwise_and(
                lax.shift_right_logical(tok, sh), jnp.int32(~7))
            colb_v = lax.bitwise_and(tok, jnp.int32((stride - 1) & ~127))
            lane = lax.iota(jnp.int32, _LANES)
            copies = []
            for j in range(_REQS_PER_WORKER):
                sel = lane == j
                r8 = pl.multiple_of(
                    jnp.max(jnp.where(sel, row8_v, jnp.int32(0))), 8)
                cb = pl.multiple_of(
                    jnp.max(jnp.where(sel, colb_v, jnp.int32(0))), 128)
                copies.append(pltpu.async_copy(
                    table_hbm.at[pl.ds(r8, 8), pl.ds(cb, 128)],
                    tiles_v.at[pl.ds(j * 8, 8), pl.ds(0, 128)], sem2))
            for cp in copies:
                cp.wait()
            row_in = lax.bitwise_and(
                lax.shift_right_logical(tok, sh), jnp.int32(7))
            col_in = lax.bitwise_and(tok, jnp.int32(127))
            tile_row = lax.bitwise_and(lane, _REQS_PER_WORKER - 1) * 8 + row_in
            vals = plsc.load_gather(tiles_v, [tile_row, col_in])
            res_v[...] = jnp.where(lens > 0, vals, jnp.int32(-1))
            pltpu.sync_copy(res_v.at[pl.ds(0, _REQS_PER_WORKER)],
                            out_hbm.at[pl.ds(base, _REQS_PER_WORKER)])

    return body(table, pool_idx, prefix_lens)


def kernel(req_to_token, req_pool_indices_tensor, prefix_lens_tensor):
    table = req_to_token.astype(jnp.int32)
    pool = req_pool_indices_tensor.astype(jnp.int32)
    lens = prefix_lens_tensor.astype(jnp.int32)
    res = _last_loc_sc(table, pool, lens)
    return res.astype(req_to_token.dtype)


# speedup vs baseline: 6.1891x; 1.0017x over previous
import functools

import jax
import jax.numpy as jnp
from jax import lax
from jax.experimental import pallas as pl
from jax.experimental.pallas import tpu as pltpu
from jax.experimental.pallas import tpu_sc as plsc

_NUM_REQS = 128
_LANES = 16
_REQS_PER_WORKER = 8
_NUM_WORKERS = _NUM_REQS // _REQS_PER_WORKER


def _last_loc_sc(table, pool_idx, prefix_lens):
    num_rows, stride = table.shape
    num_tokens = num_rows * stride
    assert stride & (stride - 1) == 0, "row stride must be a power of two"
    sh = stride.bit_length() - 1
    num_cores = 1
    mesh = plsc.VectorSubcoreMesh(
        core_axis_name="c", subcore_axis_name="s", num_cores=num_cores)

    @functools.partial(
        pl.kernel,
        mesh=mesh,
        compiler_params=pltpu.CompilerParams(
            use_tc_tiling_on_sc=True, needs_layout_passes=False),
        out_type=jax.ShapeDtypeStruct((_NUM_REQS,), jnp.int32),
        scratch_types=[
            pltpu.VMEM((_LANES,), jnp.int32),
            pltpu.VMEM((_LANES,), jnp.int32),
            pltpu.VMEM((_LANES,), jnp.int32),
            pltpu.VMEM((_REQS_PER_WORKER * 8, 128), jnp.int32),
            pltpu.SemaphoreType.DMA,
            pltpu.SemaphoreType.DMA,
        ],
    )
    def body(table_hbm, pool_hbm, len_hbm, out_hbm,
             pool_v, len_v, res_v, tiles_v, sem, sem2):
        wid = lax.axis_index("s") * num_cores + lax.axis_index("c")
        base = wid * _REQS_PER_WORKER
        cp1 = pltpu.async_copy(
            pool_hbm.at[pl.ds(base, _REQS_PER_WORKER)],
            pool_v.at[pl.ds(0, _REQS_PER_WORKER)], sem)
        cp2 = pltpu.async_copy(
            len_hbm.at[pl.ds(base, _REQS_PER_WORKER)],
            len_v.at[pl.ds(0, _REQS_PER_WORKER)], sem)
        cp1.wait()
        cp2.wait()
        lens = len_v[...]
        tok = pool_v[...] * stride + (lens - 1)
        tok = jnp.clip(tok, 0, num_tokens - 1)
        lane = lax.iota(jnp.int32, _LANES)
        copies = []
        for j in range(_REQS_PER_WORKER):
            tok_j = jnp.max(jnp.where(lane == j, tok, jnp.int32(0)))
            r8 = pl.multiple_of(
                lax.bitwise_and(lax.shift_right_logical(tok_j, sh),
                                jnp.int32(~7)), 8)
            cb = pl.multiple_of(
                lax.bitwise_and(tok_j, jnp.int32((stride - 1) & ~127)), 128)
            copies.append(pltpu.async_copy(
                table_hbm.at[pl.ds(r8, 8), pl.ds(cb, 128)],
                tiles_v.at[pl.ds(j * 8, 8), pl.ds(0, 128)], sem2))
        for cp in copies:
            cp.wait()
        row_in = lax.bitwise_and(
            lax.shift_right_logical(tok, sh), jnp.int32(7))
        col_in = lax.bitwise_and(tok, jnp.int32(127))
        tile_row = lax.bitwise_and(lane, _REQS_PER_WORKER - 1) * 8 + row_in
        vals = plsc.load_gather(tiles_v, [tile_row, col_in])
        res_v[...] = jnp.where(lens > 0, vals, jnp.int32(-1))
        pltpu.sync_copy(res_v.at[pl.ds(0, _REQS_PER_WORKER)],
                        out_hbm.at[pl.ds(base, _REQS_PER_WORKER)])

    return body(table, pool_idx, prefix_lens)


def kernel(req_to_token, req_pool_indices_tensor, prefix_lens_tensor):
    table = req_to_token.astype(jnp.int32)
    pool = req_pool_indices_tensor.astype(jnp.int32)
    lens = prefix_lens_tensor.astype(jnp.int32)
    res = _last_loc_sc(table, pool, lens)
    return res.astype(req_to_token.dtype)
